# Initial kernel scaffold; baseline (speedup 1.0000x reference)
import jax
import jax.numpy as jnp
from jax.experimental import pallas as pl


def kernel(node_feats, edge_index, edge_attr, W, b):
    raise NotImplementedError("write your pallas kernel here")


# speedup vs baseline: 1.7929x; 1.7929x over previous
import jax
import jax.numpy as jnp
from jax import lax
from jax.experimental import pallas as pl
from jax.experimental.pallas import tpu as pltpu
from jax.experimental.pallas import tpu_sc as plsc

N_NODES = 10000
N_EDGES = 160000
D_FEAT = 256
D_EDGE = 16
OUT_DIM = 512
DC = 128
NCHUNK = OUT_DIM // DC
NB = 80
EDGES_PER_TILE = N_EDGES // 16
NBATCH = EDGES_PER_TILE // NB
N_PAD = 10240
ROWS_PER_TILE = N_PAD // 16
FB = 80


def _proj_body(nf_ref, w_ref, o_ref):
    o_ref[0] = jnp.dot(nf_ref[...], w_ref[0], preferred_element_type=jnp.float32)


def _edge_body(ea_ref, w_ref, b_ref, o_ref):
    o_ref[0] = (
        jnp.dot(ea_ref[...], w_ref[0], preferred_element_type=jnp.float32)
        + b_ref[0]
    )


def _node_proj(node_feats, wq):
    nblk = 1000
    return pl.pallas_call(
        _proj_body,
        grid=(8, N_NODES // nblk),
        in_specs=[
            pl.BlockSpec((nblk, D_FEAT), lambda q, n: (n, 0)),
            pl.BlockSpec((1, D_FEAT, DC), lambda q, n: (q, 0, 0)),
        ],
        out_specs=pl.BlockSpec((1, nblk, DC), lambda q, n: (q, n, 0)),
        out_shape=jax.ShapeDtypeStruct((8, N_NODES, DC), jnp.float32),
    )(node_feats, wq)


def _edge_proj(edge_attr, we, b4):
    eblk = 8000
    return pl.pallas_call(
        _edge_body,
        grid=(NCHUNK, N_EDGES // eblk),
        in_specs=[
            pl.BlockSpec((eblk, D_EDGE), lambda c, e: (e, 0)),
            pl.BlockSpec((1, D_EDGE, DC), lambda c, e: (c, 0, 0)),
            pl.BlockSpec((1, 1, DC), lambda c, e: (c, 0, 0)),
        ],
        out_specs=pl.BlockSpec((1, eblk, DC), lambda c, e: (c, e, 0)),
        out_shape=jax.ShapeDtypeStruct((NCHUNK, N_EDGES, DC), jnp.float32),
    )(edge_attr, we, b4)


def _sc_body(p_hbm, e_hbm, src_hbm, dst_hbm, out_hbm, cnt,
             acc, sidx, gsrc, didx, gdst, ebuf, psb, pdb, cbuf,
             semg, didx2, sems):
    core = lax.axis_index("c")
    sub = lax.axis_index("s")
    r0 = sub * ROWS_PER_TILE
    zero16 = jnp.zeros((16,), jnp.float32)
    one16 = jnp.full((16,), 1.0, jnp.float32)

    def memz_psb(i, carry):
        for j in range(DC // 16):
            psb[i, pl.ds(j * 16, 16)] = zero16
        return carry

    def memone_psb(i, carry):
        for j in range(DC // 16):
            psb[i, pl.ds(j * 16, 16)] = one16
        return carry

    def zero_acc():
        lax.fori_loop(0, NB, memz_psb, 0)
        for blk in range(ROWS_PER_TILE // FB):
            pltpu.sync_copy(psb, acc.at[pl.ds(r0 + blk * FB, FB)])

    zero_acc()
    lax.fori_loop(0, NB, memone_psb, 0)
    plsc.subcore_barrier()

    def cbatch(bi, carry):
        base = sub * EDGES_PER_TILE + bi * NB
        pltpu.sync_copy(dst_hbm.at[pl.ds(base, NB)], didx)
        pltpu.sync_copy(psb, acc.at[didx], add=True)
        return carry

    lax.fori_loop(0, NBATCH, cbatch, 0)
    plsc.subcore_barrier()
    for blk in range(ROWS_PER_TILE // FB):
        r = r0 + blk * FB
        pltpu.sync_copy(acc.at[pl.ds(r, FB)], psb)

        def csqueeze(i, carry):
            cbuf[i, :] = psb[i, pl.ds(0, 16)]
            return carry

        lax.fori_loop(0, FB, csqueeze, 0)
        pltpu.sync_copy(cbuf, cnt.at[pl.ds(r, FB)])

    for k in range(2):
        c = core * 2 + k
        zero_acc()
        plsc.subcore_barrier()

        poff = c * N_NODES
        doff = (c + NCHUNK) * N_NODES
        eoff = c * N_EDGES

        def step(di, dprev, bi, first):
            base = sub * EDGES_PER_TILE + bi * NB
            pltpu.sync_copy(src_hbm.at[pl.ds(base, NB)], sidx)
            pltpu.sync_copy(dst_hbm.at[pl.ds(base, NB)], di)
            for j in range(NB // 16):
                s = pl.ds(j * 16, 16)
                gsrc[s] = sidx[s] + poff
                gdst[s] = di[s] + doff
            pltpu.async_copy(p_hbm.at[gsrc], psb, semg)
            pltpu.async_copy(p_hbm.at[gdst], pdb, semg)
            if not first:
                pltpu.make_async_copy(ebuf, acc.at[dprev], sems).wait()
            pltpu.async_copy(e_hbm.at[pl.ds(eoff + base, NB)], ebuf, semg)
            pltpu.make_async_copy(p_hbm.at[gsrc], psb, semg).wait()
            pltpu.make_async_copy(p_hbm.at[gdst], pdb, semg).wait()
            pltpu.make_async_copy(e_hbm.at[pl.ds(0, NB)], ebuf, semg).wait()

            def rbody(i, cc):
                for j in range(DC // 16):
                    s2 = pl.ds(j * 16, 16)
                    t = ebuf[i, s2] + psb[i, s2] + pdb[i, s2]
                    ebuf[i, s2] = jnp.maximum(t, 0.01 * t)
                return cc

            lax.fori_loop(0, NB, rbody, 0)
            pltpu.async_copy(ebuf, acc.at[di], sems, add=True)

        step(didx, didx2, 0, True)

        def pair(i, carry):
            step(didx2, didx, 2 * i + 1, False)
            step(didx, didx2, 2 * i + 2, False)
            return carry

        lax.fori_loop(0, (NBATCH - 1) // 2, pair, 0)
        pltpu.make_async_copy(ebuf, acc.at[didx], sems).wait()
        plsc.subcore_barrier()

        for blk in range(ROWS_PER_TILE // FB):
            r = r0 + blk * FB
            pltpu.sync_copy(acc.at[pl.ds(r, FB)], psb)
            pltpu.sync_copy(cnt.at[pl.ds(r, FB)], cbuf)

            def fin(i, cc):
                cv = jnp.maximum(cbuf[i, :], 1.0)
                for j in range(DC // 16):
                    s2 = pl.ds(j * 16, 16)
                    v = psb[i, s2] / cv
                    psb[i, s2] = jnp.where(v > 0, v, jnp.exp(v) - 1.0)
                return cc

            lax.fori_loop(0, FB, fin, 0)
            pltpu.sync_copy(psb, out_hbm.at[pl.ds(r, FB), pl.ds(c * DC, DC)])
        plsc.subcore_barrier()


def _sc_aggregate(p_flat, e_flat, src, dst):
    mesh = plsc.VectorSubcoreMesh(
        core_axis_name="c", subcore_axis_name="s", num_cores=2, num_subcores=16
    )
    return pl.kernel(
        _sc_body,
        out_type=(
            jax.ShapeDtypeStruct((N_PAD, OUT_DIM), jnp.float32),
            jax.ShapeDtypeStruct((N_PAD, 16), jnp.float32),
        ),
        mesh=mesh,
        scratch_types=[
            pltpu.VMEM_SHARED((N_PAD, DC), jnp.float32),
            pltpu.VMEM((NB,), jnp.int32),
            pltpu.VMEM((NB,), jnp.int32),
            pltpu.VMEM((NB,), jnp.int32),
            pltpu.VMEM((NB,), jnp.int32),
            pltpu.VMEM((NB, DC), jnp.float32),
            pltpu.VMEM((NB, DC), jnp.float32),
            pltpu.VMEM((NB, DC), jnp.float32),
            pltpu.VMEM((NB, 16), jnp.float32),
            pltpu.SemaphoreType.DMA,
            pltpu.VMEM((NB,), jnp.int32),
            pltpu.SemaphoreType.DMA,
        ],
    )(p_flat, e_flat, src, dst)


@jax.jit
def kernel(node_feats, edge_index, edge_attr, W, b):
    src = edge_index[0]
    dst = edge_index[1]
    ws_t = W[:, D_EDGE:D_EDGE + D_FEAT].T
    wd_t = W[:, D_EDGE + D_FEAT:].T
    wq = jnp.concatenate(
        [
            ws_t.reshape(D_FEAT, NCHUNK, DC).transpose(1, 0, 2),
            wd_t.reshape(D_FEAT, NCHUNK, DC).transpose(1, 0, 2),
        ],
        axis=0,
    )
    we = W[:, :D_EDGE].T.reshape(D_EDGE, NCHUNK, DC).transpose(1, 0, 2)
    b4 = b.reshape(NCHUNK, 1, DC)

    p8 = _node_proj(node_feats, wq)
    e4 = _edge_proj(edge_attr, we, b4)
    p_flat = p8.reshape(8 * N_NODES, DC)
    e_flat = e4.reshape(NCHUNK * N_EDGES, DC)
    out, _ = _sc_aggregate(p_flat, e_flat, src, dst)
    return out[:N_NODES]
